# baseline (speedup 1.0000x reference)
import functools

import jax
import jax.numpy as jnp
from jax import lax
from jax.experimental import pallas as pl
from jax.experimental.pallas import tpu as pltpu
from jax.experimental.pallas import tpu_sc as plsc

B = 16384
D = 64
NC = 2
NS = 16
NW = NC * NS
B_PER_W = B // NW
CHUNK = 128
NCHUNK = B_PER_W // CHUNK
NG = CHUNK // 16


def _sc_kernel(rph, rpt, rrl, rnh, rnt, ryr, pph, ppt, prl, pnh, pnt, pyr,
               ent_hbm, rel_hbm, time_hbm, out_hbm,
               idx_v, par_v, h_v, tl_v, r_v, nh_v, nt_v, t_v,
               dpT, dnT, tT, pos_v, neg_v, sem):
    wid = lax.axis_index("s") * NC + lax.axis_index("c")
    base = wid * B_PER_W

    idx_srcs = (rph, rpt, rrl, rnh, rnt, ryr)
    par_srcs = (pph, ppt, prl, pnh, pnt, pyr)
    tables = (ent_hbm, ent_hbm, rel_hbm, ent_hbm, ent_hbm, time_hbm)
    bufs = (h_v, tl_v, r_v, nh_v, nt_v, t_v)
    iota = lax.iota(jnp.int32, 16)

    for c in range(NCHUNK):
        off = base + c * CHUNK
        for j in range(6):
            pltpu.sync_copy(idx_srcs[j].at[pl.ds(off, CHUNK)], idx_v.at[j])
            pltpu.sync_copy(par_srcs[j].at[pl.ds(off, CHUNK)], par_v.at[j])
        cps = [pltpu.async_copy(tables[j].at[idx_v.at[j]], bufs[j], sem)
               for j in range(6)]
        for cp in cps:
            cp.wait()

        def group(g, _):
            rows = jnp.full((16,), g * 16, jnp.int32) + iota
            sl16 = pl.ds(g * 16, 16)
            pars = [par_v[j, sl16] for j in range(6)]

            def pass1(d, carry):
                ip_p, ip_n = carry
                dvec = jnp.full((16,), d, jnp.int32)
                h = plsc.load_gather(h_v, [rows, pars[0] + dvec])
                tl = plsc.load_gather(tl_v, [rows, pars[1] + dvec])
                r = plsc.load_gather(r_v, [rows, pars[2] + dvec])
                nh = plsc.load_gather(nh_v, [rows, pars[3] + dvec])
                nt = plsc.load_gather(nt_v, [rows, pars[4] + dvec])
                t = plsc.load_gather(t_v, [rows, pars[5] + dvec])
                dp = h + r - tl
                dn = nh + r - nt
                dpT[d, :] = dp
                dnT[d, :] = dn
                tT[d, :] = t
                return ip_p + dp * t, ip_n + dn * t

            zero = jnp.zeros((16,), jnp.float32)
            ip_p, ip_n = lax.fori_loop(0, D, pass1, (zero, zero), unroll=8)

            def pass2(d, carry):
                ap, an = carry
                dp = dpT[d, :]
                dn = dnT[d, :]
                t = tT[d, :]
                ap = ap + jnp.abs(dp - t * ip_p)
                an = an + jnp.abs(dn - t * ip_n)
                return ap, an

            ap, an = lax.fori_loop(0, D, pass2, (zero, zero), unroll=8)
            pos_v[sl16] = ap
            neg_v[sl16] = an
            return 0

        lax.fori_loop(0, NG, group, 0)
        pltpu.sync_copy(pos_v, out_hbm.at[0, pl.ds(off, CHUNK)])
        pltpu.sync_copy(neg_v, out_hbm.at[1, pl.ds(off, CHUNK)])


@jax.jit
def _run(ph, pt, rl, nh, nt, yr, ent, rel, time):
    mesh = plsc.VectorSubcoreMesh(core_axis_name="c", subcore_axis_name="s")
    kfn = functools.partial(
        pl.kernel,
        mesh=mesh,
        compiler_params=pltpu.CompilerParams(needs_layout_passes=False),
        out_type=jax.ShapeDtypeStruct((2, B), jnp.float32),
        scratch_types=[
            pltpu.VMEM((6, CHUNK), jnp.int32),
            pltpu.VMEM((6, CHUNK), jnp.int32),
            pltpu.VMEM((CHUNK, 2 * D), jnp.float32),
            pltpu.VMEM((CHUNK, 2 * D), jnp.float32),
            pltpu.VMEM((CHUNK, 2 * D), jnp.float32),
            pltpu.VMEM((CHUNK, 2 * D), jnp.float32),
            pltpu.VMEM((CHUNK, 2 * D), jnp.float32),
            pltpu.VMEM((CHUNK, 2 * D), jnp.float32),
            pltpu.VMEM((D, 16), jnp.float32),
            pltpu.VMEM((D, 16), jnp.float32),
            pltpu.VMEM((D, 16), jnp.float32),
            pltpu.VMEM((CHUNK,), jnp.float32),
            pltpu.VMEM((CHUNK,), jnp.float32),
            pltpu.SemaphoreType.DMA,
        ],
    )(_sc_kernel)

    ent2 = ent.reshape(-1, 2 * D)
    rel2 = rel.reshape(-1, 2 * D)
    tim2 = time.reshape(-1, 2 * D)
    idxs = (ph, pt, rl, nh, nt, yr)
    rows = [i >> 1 for i in idxs]
    pars = [(i & 1) << 6 for i in idxs]
    return kfn(*rows, *pars, ent2, rel2, tim2)


def kernel(pos_head, pos_tail, rel, neg_head, neg_tail, start_year,
           ent_embeddings, rel_embeddings, time_embeddings):
    ph = pos_head.reshape(B)
    pt = pos_tail.reshape(B)
    rl = rel.reshape(B)
    nh = neg_head.reshape(B)
    nt = neg_tail.reshape(B)
    return _run(ph, pt, rl, nh, nt, start_year,
                ent_embeddings, rel_embeddings, time_embeddings)
